# baseline (speedup 1.0000x reference)
import functools

import jax
import jax.numpy as jnp
from jax import lax
from jax.experimental import pallas as pl
from jax.experimental.pallas import tpu as pltpu
from jax.experimental.pallas import tpu_sc as plsc

VOCAB = 1000000
MAX_LEN = 200
EMBED_DIM = 64
BATCH = 4096

NC = 2
NS = 16
NW = NC * NS

ROWS = BATCH * MAX_LEN
ROWS_PER_W = ROWS // NW
SEQ_PER_CHUNK = 4
CHUNK = SEQ_PER_CHUNK * MAX_LEN
N_CHUNKS = ROWS_PER_W // CHUNK
GROUP = 80
N_GROUPS = CHUNK // GROUP
LANES = 16
VPR = EMBED_DIM // LANES


@functools.partial(
    pl.kernel,
    mesh=plsc.VectorSubcoreMesh(core_axis_name="c", subcore_axis_name="s"),
    out_type=jax.ShapeDtypeStruct((ROWS, EMBED_DIM), jnp.float32),
    scratch_types=[
        pltpu.VMEM((MAX_LEN, EMBED_DIM), jnp.float32),
        pltpu.VMEM((CHUNK,), jnp.int32),
        pltpu.VMEM((CHUNK, EMBED_DIM), jnp.float32),
        pltpu.SemaphoreType.DMA,
    ],
    compiler_params=pltpu.CompilerParams(use_tc_tiling_on_sc=False),
)
def _embed(idx_hbm, table_hbm, pos_hbm, out_hbm, pos_v, idx_v, rows_v, sem):
    wid = lax.axis_index("s") * NC + lax.axis_index("c")
    base = wid * ROWS_PER_W
    pltpu.sync_copy(pos_hbm, pos_v)

    def chunk_body(ci, carry):
        cbase = base + ci * CHUNK
        pltpu.sync_copy(idx_hbm.at[pl.ds(cbase, CHUNK)], idx_v)
        copies = [
            pltpu.async_copy(
                table_hbm.at[idx_v.at[pl.ds(g * GROUP, GROUP)]],
                rows_v.at[pl.ds(g * GROUP, GROUP), :],
                sem,
            )
            for g in range(N_GROUPS)
        ]
        for cp in copies:
            cp.wait()

        def add_pos(l, c):
            for j in range(VPR):
                pv = pos_v[l, pl.ds(j * LANES, LANES)]
                for s in range(SEQ_PER_CHUNK):
                    r = s * MAX_LEN + l
                    rows_v[r, pl.ds(j * LANES, LANES)] = (
                        rows_v[r, pl.ds(j * LANES, LANES)] + pv
                    )
            return c

        lax.fori_loop(0, MAX_LEN, add_pos, 0)
        pltpu.sync_copy(rows_v, out_hbm.at[pl.ds(cbase, CHUNK)])
        return carry

    lax.fori_loop(0, N_CHUNKS, chunk_body, 0)


def kernel(inputs, token_table, pos_table):
    idx = inputs.reshape(-1).astype(jnp.int32)
    out = _embed(idx, token_table, pos_table)
    return out.reshape(BATCH, MAX_LEN, EMBED_DIM)


# speedup vs baseline: 1.0012x; 1.0012x over previous
import functools

import jax
import jax.numpy as jnp
from jax import lax
from jax.experimental import pallas as pl
from jax.experimental.pallas import tpu as pltpu
from jax.experimental.pallas import tpu_sc as plsc

VOCAB = 1000000
MAX_LEN = 200
EMBED_DIM = 64
BATCH = 4096

NC = 2
NS = 16
NW = NC * NS

B_PER_W = BATCH // NW
SEQ_PER_CHUNK = 4
CHUNK = SEQ_PER_CHUNK * MAX_LEN
N_CHUNKS = B_PER_W // SEQ_PER_CHUNK
LANES = 16
VPR = EMBED_DIM // LANES


@functools.partial(
    pl.kernel,
    mesh=plsc.VectorSubcoreMesh(core_axis_name="c", subcore_axis_name="s"),
    out_type=jax.ShapeDtypeStruct((BATCH, MAX_LEN, EMBED_DIM), jnp.float32),
    scratch_types=[
        pltpu.VMEM((MAX_LEN, EMBED_DIM), jnp.float32),
        pltpu.VMEM((CHUNK,), jnp.int32),
        pltpu.VMEM((SEQ_PER_CHUNK, MAX_LEN, EMBED_DIM), jnp.float32),
        pltpu.SemaphoreType.DMA,
    ],
    compiler_params=pltpu.CompilerParams(use_tc_tiling_on_sc=False),
)
def _embed(idx_hbm, table_hbm, pos_hbm, out_hbm, pos_v, idx_v, rows_v, sem):
    wid = lax.axis_index("s") * NC + lax.axis_index("c")
    b_base = wid * B_PER_W
    pltpu.sync_copy(pos_hbm, pos_v)

    def chunk_body(ci, carry):
        bb = b_base + ci * SEQ_PER_CHUNK
        pltpu.sync_copy(idx_hbm.at[pl.ds(bb * MAX_LEN, CHUNK)], idx_v)
        copies = [
            pltpu.async_copy(
                table_hbm.at[idx_v.at[pl.ds(s * MAX_LEN, MAX_LEN)]],
                rows_v.at[s],
                sem,
            )
            for s in range(SEQ_PER_CHUNK)
        ]
        for cp in copies:
            cp.wait()

        def add_pos(l, c):
            for j in range(VPR):
                pv = pos_v[l, pl.ds(j * LANES, LANES)]
                for s in range(SEQ_PER_CHUNK):
                    rows_v[s, l, pl.ds(j * LANES, LANES)] = (
                        rows_v[s, l, pl.ds(j * LANES, LANES)] + pv
                    )
            return c

        lax.fori_loop(0, MAX_LEN, add_pos, 0)
        pltpu.sync_copy(rows_v, out_hbm.at[pl.ds(bb, SEQ_PER_CHUNK)])
        return carry

    lax.fori_loop(0, N_CHUNKS, chunk_body, 0)


def kernel(inputs, token_table, pos_table):
    idx = inputs.reshape(-1).astype(jnp.int32)
    return _embed(idx, token_table, pos_table)
